# baseline (speedup 1.0000x reference)
import functools

import jax
import jax.numpy as jnp
from jax import lax
from jax.experimental import pallas as pl
from jax.experimental.pallas import tpu as pltpu
from jax.experimental.pallas import tpu_sc as plsc

D0, D1, D2 = 160, 160, 160
C = 12
CPAD = 16
NROWS = D0 * D1 * D2
NWORKERS = 32
CHUNK = 256
NIDX = 8 * CHUNK
GATHER_SLICE = 128
BX = 16384


def _prep_table(grid):
    g2 = grid.reshape(C, NROWS)

    def body(g_ref, out_ref):
        eye = (lax.broadcasted_iota(jnp.int32, (C, CPAD), 0)
               == lax.broadcasted_iota(jnp.int32, (C, CPAD), 1))
        out_ref[...] = lax.dot_general(
            g_ref[...], eye.astype(jnp.float32), (((0,), (0,)), ((), ())),
            preferred_element_type=jnp.float32)

    return pl.pallas_call(
        body,
        grid=(NROWS // BX,),
        in_specs=[pl.BlockSpec((C, BX), lambda i: (0, i))],
        out_specs=pl.BlockSpec((BX, CPAD), lambda i: (i, 0)),
        out_shape=jax.ShapeDtypeStruct((NROWS, CPAD), jnp.float32),
    )(g2)


def _build_sc_kernel(n_pts):
    ppw = n_pts // NWORKERS
    nchunks = ppw // CHUNK
    nh = nchunks // 2
    mesh = plsc.VectorSubcoreMesh(core_axis_name="c", subcore_axis_name="s")

    @functools.partial(
        pl.kernel,
        mesh=mesh,
        compiler_params=pltpu.CompilerParams(
            needs_layout_passes=False, use_tc_tiling_on_sc=False),
        out_type=jax.ShapeDtypeStruct((n_pts * C,), jnp.float32),
        scratch_types=[
            pltpu.VMEM((8, 16), jnp.float32),
            pltpu.VMEM((3 * CHUNK,), jnp.float32),
            pltpu.VMEM((2, NIDX), jnp.int32),
            pltpu.VMEM((2, NIDX), jnp.float32),
            pltpu.VMEM((2, NIDX, CPAD), jnp.float32),
            pltpu.VMEM((CHUNK * C,), jnp.float32),
            pltpu.SemaphoreType.DMA,
            pltpu.SemaphoreType.DMA,
        ],
    )
    def k(q_hbm, table_hbm, consts_hbm, out_hbm,
          consts_v, q_v, idx_v, w_v, rows_v, out_v, sem0, sem1):
        cid = lax.axis_index("c")
        sid = lax.axis_index("s")
        wid = sid * 2 + cid
        base0 = wid * ppw

        pltpu.sync_copy(consts_hbm, consts_v)
        lane = lax.iota(jnp.int32, 16)
        lane3 = lane * 3
        laneC = lane * C
        sx = consts_v[0, :]
        sy = consts_v[1, :]
        sz = consts_v[2, :]
        ox = consts_v[3, :]
        oy = consts_v[4, :]
        oz = consts_v[5, :]
        sems = (sem0, sem1)

        def produce(ci, p):
            base = base0 + ci * CHUNK
            pltpu.sync_copy(q_hbm.at[pl.ds(base * 3, 3 * CHUNK)], q_v)

            def grp_a(g, _):
                off3 = g * 48
                x = plsc.load_gather(q_v, [lane3 + off3])
                y = plsc.load_gather(q_v, [lane3 + (off3 + 1)])
                z = plsc.load_gather(q_v, [lane3 + (off3 + 2)])
                qx = x * sx + ox
                qy = y * sy + oy
                qz = z * sz + oz
                ix = jnp.minimum(jnp.maximum(qx.astype(jnp.int32), 0), D0 - 2)
                iy = jnp.minimum(jnp.maximum(qy.astype(jnp.int32), 0), D1 - 2)
                iz = jnp.minimum(jnp.maximum(qz.astype(jnp.int32), 0), D2 - 2)
                fx = qx - ix.astype(jnp.float32)
                fy = qy - iy.astype(jnp.float32)
                fz = qz - iz.astype(jnp.float32)
                flat = ix * (D1 * D2) + iy * D2 + iz
                gx0 = 1.0 - fx
                gy0 = 1.0 - fy
                gz0 = 1.0 - fz
                wxy = (gx0 * gy0, gx0 * fy, fx * gy0, fx * fy)
                p16 = g * 16
                for kk in range(8):
                    dx, dy, dz = kk >> 2, (kk >> 1) & 1, kk & 1
                    corner_off = dx * (D1 * D2) + dy * D2 + dz
                    idx_v[p, pl.ds(kk * CHUNK + p16, 16)] = flat + corner_off
                    wz = fz if dz else gz0
                    w_v[p, pl.ds(kk * CHUNK + p16, 16)] = wxy[2 * dx + dy] * wz
                return 0

            lax.fori_loop(0, CHUNK // 16, grp_a, 0)
            for j in range(NIDX // GATHER_SLICE):
                pltpu.async_copy(
                    table_hbm.at[idx_v.at[p, pl.ds(j * GATHER_SLICE,
                                                   GATHER_SLICE)]],
                    rows_v.at[p, pl.ds(j * GATHER_SLICE, GATHER_SLICE), :],
                    sems[p])

        def consume(ci, p):
            base = base0 + ci * CHUNK
            pltpu.make_async_copy(
                table_hbm.at[pl.ds(0, NIDX), :], rows_v.at[p], sems[p]).wait()

            def grp_c(g, _):
                p16 = g * 16
                ws = [w_v[p, pl.ds(kk * CHUNK + p16, 16)] for kk in range(8)]
                row0 = lane + p16
                for ch in range(C):
                    col = jnp.full((16,), ch, jnp.int32)
                    acc = ws[0] * plsc.load_gather(rows_v.at[p], [row0, col])
                    for kk in range(1, 8):
                        v = plsc.load_gather(
                            rows_v.at[p], [row0 + kk * CHUNK, col])
                        acc = acc + ws[kk] * v
                    plsc.store_scatter(out_v, [laneC + (p16 * C + ch)], acc)
                return 0

            lax.fori_loop(0, CHUNK // 16, grp_c, 0)
            pltpu.sync_copy(out_v, out_hbm.at[pl.ds(base * C, CHUNK * C)])

        produce(0, 0)

        def pair_body(i, _):
            ci = 2 * i
            produce(ci + 1, 1)
            consume(ci, 0)

            @pl.when(i < nh - 1)
            def _():
                produce(ci + 2, 0)

            consume(ci + 1, 1)
            return 0

        lax.fori_loop(0, nh, pair_body, 0)

    return k


def kernel(xyz, grid, xyz_min, xyz_max):
    shape = xyz.shape[:-1]
    pts = xyz.reshape(-1, 3)
    n_pts = pts.shape[0]

    table = _prep_table(grid)

    sizes = jnp.array([D0 - 1, D1 - 1, D2 - 1], dtype=jnp.float32)
    scale = sizes / (xyz_max - xyz_min)
    off = -xyz_min * scale
    consts = jnp.zeros((8, 16), jnp.float32)
    consts = consts.at[0:3, :].set(jnp.broadcast_to(scale[:, None], (3, 16)))
    consts = consts.at[3:6, :].set(jnp.broadcast_to(off[:, None], (3, 16)))

    q_flat = pts.reshape(-1)
    out = _build_sc_kernel(n_pts)(q_flat, table, consts)
    out = out.reshape(*shape, C)
    return out


# speedup vs baseline: 1.3134x; 1.3134x over previous
import functools

import jax
import jax.numpy as jnp
from jax import lax
from jax.experimental import pallas as pl
from jax.experimental.pallas import tpu as pltpu
from jax.experimental.pallas import tpu_sc as plsc

D0, D1, D2 = 160, 160, 160
C = 12
CPAD = 16
NROWS = D0 * D1 * D2
NWORKERS = 32
CHUNK = 256
NIDX = 8 * CHUNK
GATHER_SLICE = 128


def _prep_table(grid):
    BY = 80

    def body(g_ref, out_ref):
        v = g_ref[...].reshape(C, BY * D2)
        eye = (lax.broadcasted_iota(jnp.int32, (C, CPAD), 0)
               == lax.broadcasted_iota(jnp.int32, (C, CPAD), 1))
        out_ref[...] = lax.dot_general(
            v, eye.astype(jnp.float32), (((0,), (0,)), ((), ())),
            preferred_element_type=jnp.float32,
            precision=lax.Precision.HIGHEST)

    nj = D1 // BY
    return pl.pallas_call(
        body,
        grid=(D0 * nj,),
        in_specs=[pl.BlockSpec((C, 1, BY, D2),
                               lambda i: (0, i // nj, i % nj, 0))],
        out_specs=pl.BlockSpec((BY * D2, CPAD), lambda i: (i, 0)),
        out_shape=jax.ShapeDtypeStruct((NROWS, CPAD), jnp.float32),
    )(grid)


def _build_sc_kernel(n_pts):
    ppw = n_pts // NWORKERS
    nchunks = ppw // CHUNK
    nh = nchunks // 2
    mesh = plsc.VectorSubcoreMesh(core_axis_name="c", subcore_axis_name="s")

    @functools.partial(
        pl.kernel,
        mesh=mesh,
        compiler_params=pltpu.CompilerParams(
            needs_layout_passes=False, use_tc_tiling_on_sc=False),
        out_type=jax.ShapeDtypeStruct((C, n_pts), jnp.float32),
        scratch_types=[
            pltpu.VMEM((8, 16), jnp.float32),
            pltpu.VMEM((3 * CHUNK,), jnp.float32),
            pltpu.VMEM((2, NIDX), jnp.int32),
            pltpu.VMEM((2, NIDX), jnp.float32),
            pltpu.VMEM((2, NIDX, CPAD), jnp.float32),
            pltpu.VMEM((C, CHUNK), jnp.float32),
            pltpu.SemaphoreType.DMA,
            pltpu.SemaphoreType.DMA,
        ],
    )
    def k(q_hbm, table_hbm, consts_hbm, out_hbm,
          consts_v, q_v, idx_v, w_v, rows_v, out_v, sem0, sem1):
        cid = lax.axis_index("c")
        sid = lax.axis_index("s")
        wid = sid * 2 + cid
        base0 = wid * ppw

        pltpu.sync_copy(consts_hbm, consts_v)
        lane = lax.iota(jnp.int32, 16)
        lane3 = lane * 3
        sx = consts_v[0, :]
        sy = consts_v[1, :]
        sz = consts_v[2, :]
        ox = consts_v[3, :]
        oy = consts_v[4, :]
        oz = consts_v[5, :]
        sems = (sem0, sem1)

        def produce(ci, p):
            base = base0 + ci * CHUNK
            pltpu.sync_copy(q_hbm.at[pl.ds(base * 3, 3 * CHUNK)], q_v)

            def grp_a(g, _):
                off3 = g * 48
                x = plsc.load_gather(q_v, [lane3 + off3])
                y = plsc.load_gather(q_v, [lane3 + (off3 + 1)])
                z = plsc.load_gather(q_v, [lane3 + (off3 + 2)])
                qx = x * sx + ox
                qy = y * sy + oy
                qz = z * sz + oz
                ix = jnp.minimum(jnp.maximum(qx.astype(jnp.int32), 0), D0 - 2)
                iy = jnp.minimum(jnp.maximum(qy.astype(jnp.int32), 0), D1 - 2)
                iz = jnp.minimum(jnp.maximum(qz.astype(jnp.int32), 0), D2 - 2)
                fx = qx - ix.astype(jnp.float32)
                fy = qy - iy.astype(jnp.float32)
                fz = qz - iz.astype(jnp.float32)
                flat = ix * (D1 * D2) + iy * D2 + iz
                gx0 = 1.0 - fx
                gy0 = 1.0 - fy
                gz0 = 1.0 - fz
                wxy = (gx0 * gy0, gx0 * fy, fx * gy0, fx * fy)
                p16 = g * 16
                for kk in range(8):
                    dx, dy, dz = kk >> 2, (kk >> 1) & 1, kk & 1
                    corner_off = dx * (D1 * D2) + dy * D2 + dz
                    idx_v[p, pl.ds(kk * CHUNK + p16, 16)] = flat + corner_off
                    wz = fz if dz else gz0
                    w_v[p, pl.ds(kk * CHUNK + p16, 16)] = wxy[2 * dx + dy] * wz
                return 0

            lax.fori_loop(0, CHUNK // 16, grp_a, 0)
            for j in range(NIDX // GATHER_SLICE):
                pltpu.async_copy(
                    table_hbm.at[idx_v.at[p, pl.ds(j * GATHER_SLICE,
                                                   GATHER_SLICE)]],
                    rows_v.at[p, pl.ds(j * GATHER_SLICE, GATHER_SLICE), :],
                    sems[p])

        def consume(ci, p):
            base = base0 + ci * CHUNK
            pltpu.make_async_copy(
                table_hbm.at[pl.ds(0, NIDX), :], rows_v.at[p], sems[p]).wait()

            def grp_c(g, _):
                p16 = g * 16
                ws = [w_v[p, pl.ds(kk * CHUNK + p16, 16)] for kk in range(8)]
                row0 = lane + p16
                for ch in range(C):
                    col = jnp.full((16,), ch, jnp.int32)
                    acc = ws[0] * plsc.load_gather(rows_v.at[p], [row0, col])
                    for kk in range(1, 8):
                        v = plsc.load_gather(
                            rows_v.at[p], [row0 + kk * CHUNK, col])
                        acc = acc + ws[kk] * v
                    out_v[ch, pl.ds(p16, 16)] = acc
                return 0

            lax.fori_loop(0, CHUNK // 16, grp_c, 0)
            pltpu.sync_copy(out_v, out_hbm.at[:, pl.ds(base, CHUNK)])

        produce(0, 0)

        def pair_body(i, _):
            ci = 2 * i
            produce(ci + 1, 1)
            consume(ci, 0)

            @pl.when(i < nh - 1)
            def _():
                produce(ci + 2, 0)

            consume(ci + 1, 1)
            return 0

        lax.fori_loop(0, nh, pair_body, 0)

    return k


def kernel(xyz, grid, xyz_min, xyz_max):
    shape = xyz.shape[:-1]
    pts = xyz.reshape(-1, 3)
    n_pts = pts.shape[0]

    table = _prep_table(grid)

    sizes = jnp.array([D0 - 1, D1 - 1, D2 - 1], dtype=jnp.float32)
    scale = sizes / (xyz_max - xyz_min)
    off = -xyz_min * scale
    consts = jnp.zeros((8, 16), jnp.float32)
    consts = consts.at[0:3, :].set(jnp.broadcast_to(scale[:, None], (3, 16)))
    consts = consts.at[3:6, :].set(jnp.broadcast_to(off[:, None], (3, 16)))

    q_flat = pts.reshape(-1)
    out_cm = _build_sc_kernel(n_pts)(q_flat, table, consts)
    out = out_cm.T.reshape(*shape, C)
    return out


# speedup vs baseline: 1.4963x; 1.1392x over previous
import functools

import jax
import jax.numpy as jnp
from jax import lax
from jax.experimental import pallas as pl
from jax.experimental.pallas import tpu as pltpu
from jax.experimental.pallas import tpu_sc as plsc

D0, D1, D2 = 160, 160, 160
C = 12
CPAD = 16
NROWS = D0 * D1 * D2
NWORKERS = 32
CHUNK = 256
NIDX = 8 * CHUNK
GATHER_SLICE = 128


YS = 16
TROWS = NROWS * CPAD // 128


def _repack_table(grid):
    mesh = plsc.VectorSubcoreMesh(core_axis_name="c", subcore_axis_name="s")
    xpw = D0 // NWORKERS
    nslab = D1 // YS

    @functools.partial(
        pl.kernel,
        mesh=mesh,
        compiler_params=pltpu.CompilerParams(
            needs_layout_passes=False, use_tc_tiling_on_sc=True),
        out_type=jax.ShapeDtypeStruct((TROWS, 128), jnp.float32),
        scratch_types=[
            pltpu.VMEM((C, YS, 128), jnp.float32),
            pltpu.VMEM((C, YS, 32), jnp.float32),
            pltpu.VMEM((YS * D2 * CPAD // 128, 128), jnp.float32),
            pltpu.SemaphoreType.DMA,
        ],
    )
    def k(g_hbm, out_hbm, buf_a, buf_b, ob, sem):
        cid = lax.axis_index("c")
        sid = lax.axis_index("s")
        wid = sid * 2 + cid
        lane = lax.iota(jnp.int32, 16)
        lane_hi = lane >> 3
        lane_lo16 = (lane & 7) * 16

        def slab_body(i, _):
            x = wid * xpw + i // nslab
            y0 = (i % nslab) * YS

            def fire(c, _):
                pltpu.async_copy(
                    g_hbm.at[c, x, pl.ds(y0, YS), pl.ds(0, 128)],
                    buf_a.at[c], sem)
                pltpu.async_copy(
                    g_hbm.at[c, x, pl.ds(y0, YS), pl.ds(128, 32)],
                    buf_b.at[c], sem)
                return 0

            lax.fori_loop(0, C, fire, 0)
            pltpu.make_async_copy(
                g_hbm.at[pl.ds(0, C), 0, pl.ds(0, YS), pl.ds(0, 128)],
                buf_a, sem).wait()
            pltpu.make_async_copy(
                g_hbm.at[pl.ds(0, C), 0, pl.ds(0, YS), pl.ds(128, 32)],
                buf_b, sem).wait()

            def chan_body(cy, _):
                c = cy // YS
                yy = cy % YS
                colv = lane_lo16 + c
                rbase = yy * (D2 * CPAD // 128)
                for zg in range(8):
                    v = buf_a[c, yy, pl.ds(zg * 16, 16)]
                    rowv = lane_hi + (rbase + zg * 2)
                    plsc.store_scatter(ob, [rowv, colv], v)
                for zg in range(2):
                    v = buf_b[c, yy, pl.ds(zg * 16, 16)]
                    rowv = lane_hi + (rbase + 16 + zg * 2)
                    plsc.store_scatter(ob, [rowv, colv], v)
                return 0

            lax.fori_loop(0, C * YS, chan_body, 0)
            row0 = (x * D1 + y0) * (D2 * CPAD // 128)
            pltpu.sync_copy(ob, out_hbm.at[pl.ds(row0, YS * D2 * CPAD // 128), :])
            return 0

        lax.fori_loop(0, xpw * nslab, slab_body, 0)

    return k(grid)


def _build_sc_kernel(n_pts):
    ppw = n_pts // NWORKERS
    nchunks = ppw // CHUNK
    nh = nchunks // 2
    mesh = plsc.VectorSubcoreMesh(core_axis_name="c", subcore_axis_name="s")

    @functools.partial(
        pl.kernel,
        mesh=mesh,
        compiler_params=pltpu.CompilerParams(
            needs_layout_passes=False, use_tc_tiling_on_sc=False),
        out_type=jax.ShapeDtypeStruct((C, n_pts), jnp.float32),
        scratch_types=[
            pltpu.VMEM((8, 16), jnp.float32),
            pltpu.VMEM((3 * CHUNK,), jnp.float32),
            pltpu.VMEM((2, NIDX), jnp.int32),
            pltpu.VMEM((2, NIDX), jnp.float32),
            pltpu.VMEM((2, NIDX, CPAD), jnp.float32),
            pltpu.VMEM((C, CHUNK), jnp.float32),
            pltpu.SemaphoreType.DMA,
            pltpu.SemaphoreType.DMA,
        ],
    )
    def k(q_hbm, table_hbm, consts_hbm, out_hbm,
          consts_v, q_v, idx_v, w_v, rows_v, out_v, sem0, sem1):
        cid = lax.axis_index("c")
        sid = lax.axis_index("s")
        wid = sid * 2 + cid
        base0 = wid * ppw

        pltpu.sync_copy(consts_hbm, consts_v)
        lane = lax.iota(jnp.int32, 16)
        lane3 = lane * 3
        sx = consts_v[0, :]
        sy = consts_v[1, :]
        sz = consts_v[2, :]
        ox = consts_v[3, :]
        oy = consts_v[4, :]
        oz = consts_v[5, :]
        sems = (sem0, sem1)

        def produce(ci, p):
            base = base0 + ci * CHUNK
            pltpu.sync_copy(q_hbm.at[pl.ds(base * 3, 3 * CHUNK)], q_v)

            def grp_a(g, _):
                off3 = g * 48
                x = plsc.load_gather(q_v, [lane3 + off3])
                y = plsc.load_gather(q_v, [lane3 + (off3 + 1)])
                z = plsc.load_gather(q_v, [lane3 + (off3 + 2)])
                qx = x * sx + ox
                qy = y * sy + oy
                qz = z * sz + oz
                ix = jnp.minimum(jnp.maximum(qx.astype(jnp.int32), 0), D0 - 2)
                iy = jnp.minimum(jnp.maximum(qy.astype(jnp.int32), 0), D1 - 2)
                iz = jnp.minimum(jnp.maximum(qz.astype(jnp.int32), 0), D2 - 2)
                fx = qx - ix.astype(jnp.float32)
                fy = qy - iy.astype(jnp.float32)
                fz = qz - iz.astype(jnp.float32)
                flat = ix * (D1 * D2) + iy * D2 + iz
                gx0 = 1.0 - fx
                gy0 = 1.0 - fy
                gz0 = 1.0 - fz
                wxy = (gx0 * gy0, gx0 * fy, fx * gy0, fx * fy)
                p16 = g * 16
                for kk in range(8):
                    dx, dy, dz = kk >> 2, (kk >> 1) & 1, kk & 1
                    corner_off = dx * (D1 * D2) + dy * D2 + dz
                    idx_v[p, pl.ds(kk * CHUNK + p16, 16)] = flat + corner_off
                    wz = fz if dz else gz0
                    w_v[p, pl.ds(kk * CHUNK + p16, 16)] = wxy[2 * dx + dy] * wz
                return 0

            lax.fori_loop(0, CHUNK // 16, grp_a, 0)
            for j in range(NIDX // GATHER_SLICE):
                pltpu.async_copy(
                    table_hbm.at[idx_v.at[p, pl.ds(j * GATHER_SLICE,
                                                   GATHER_SLICE)]],
                    rows_v.at[p, pl.ds(j * GATHER_SLICE, GATHER_SLICE), :],
                    sems[p])

        def consume(ci, p):
            base = base0 + ci * CHUNK
            pltpu.make_async_copy(
                table_hbm.at[pl.ds(0, NIDX), :], rows_v.at[p], sems[p]).wait()

            def grp_c(g, _):
                p16 = g * 16
                ws = [w_v[p, pl.ds(kk * CHUNK + p16, 16)] for kk in range(8)]
                row0 = lane + p16
                for ch in range(C):
                    col = jnp.full((16,), ch, jnp.int32)
                    acc = ws[0] * plsc.load_gather(rows_v.at[p], [row0, col])
                    for kk in range(1, 8):
                        v = plsc.load_gather(
                            rows_v.at[p], [row0 + kk * CHUNK, col])
                        acc = acc + ws[kk] * v
                    out_v[ch, pl.ds(p16, 16)] = acc
                return 0

            lax.fori_loop(0, CHUNK // 16, grp_c, 0)
            pltpu.sync_copy(out_v, out_hbm.at[:, pl.ds(base, CHUNK)])

        produce(0, 0)

        def pair_body(i, _):
            ci = 2 * i
            produce(ci + 1, 1)
            consume(ci, 0)

            @pl.when(i < nh - 1)
            def _():
                produce(ci + 2, 0)

            consume(ci + 1, 1)
            return 0

        lax.fori_loop(0, nh, pair_body, 0)

    return k


def kernel(xyz, grid, xyz_min, xyz_max):
    shape = xyz.shape[:-1]
    pts = xyz.reshape(-1, 3)
    n_pts = pts.shape[0]

    table = _repack_table(grid).reshape(NROWS, CPAD)

    sizes = jnp.array([D0 - 1, D1 - 1, D2 - 1], dtype=jnp.float32)
    scale = sizes / (xyz_max - xyz_min)
    off = -xyz_min * scale
    consts = jnp.zeros((8, 16), jnp.float32)
    consts = consts.at[0:3, :].set(jnp.broadcast_to(scale[:, None], (3, 16)))
    consts = consts.at[3:6, :].set(jnp.broadcast_to(off[:, None], (3, 16)))

    q_flat = pts.reshape(-1)
    out_cm = _build_sc_kernel(n_pts)(q_flat, table, consts)
    out = out_cm.T.reshape(*shape, C)
    return out


# speedup vs baseline: 3.1275x; 2.0902x over previous
import functools

import jax
import jax.numpy as jnp
from jax import lax
from jax.experimental import pallas as pl
from jax.experimental.pallas import tpu as pltpu
from jax.experimental.pallas import tpu_sc as plsc

D0, D1, D2 = 160, 160, 160
C = 12
CPAD = 16
NPAIR = C // 2
NROWS = D0 * D1 * D2
NWORKERS = 32
CHUNK = 512
NIDX = 4 * CHUNK
GATHER_SLICE = 128

YS = 16
TROWS = NROWS * CPAD // 128


def _repack_table(grid):
    mesh = plsc.VectorSubcoreMesh(core_axis_name="c", subcore_axis_name="s")
    xpw = D0 // NWORKERS
    nslab = D1 // YS
    obr = YS * D2 * CPAD // 128

    @functools.partial(
        pl.kernel,
        mesh=mesh,
        compiler_params=pltpu.CompilerParams(
            needs_layout_passes=False, use_tc_tiling_on_sc=True),
        out_type=jax.ShapeDtypeStruct((TROWS, 128), jnp.float32),
        scratch_types=[
            pltpu.VMEM((C, YS, 128), jnp.float32),
            pltpu.VMEM((C, YS, 32), jnp.float32),
            pltpu.VMEM((YS * D2 * CPAD // 128, 128), jnp.float32),
            pltpu.SemaphoreType.DMA,
        ],
    )
    def k(g_hbm, out_hbm, buf_a, buf_b, ob, sem):
        cid = lax.axis_index("c")
        sid = lax.axis_index("s")
        wid = sid * 2 + cid
        lane = lax.iota(jnp.int32, 16)
        lane_hi = lane >> 3
        lane_lo16 = (lane & 7) * 16
        rp2 = [(lane + (zg * 16 - 1)) >> 3 for zg in range(10)]
        cp2 = [((lane + (zg * 16 - 1)) & 7) * 16 for zg in range(10)]
        m0 = lane >= 1

        def slab_body(i, _):
            x = wid * xpw + i // nslab
            y0 = (i % nslab) * YS

            def fire(c, _):
                pltpu.async_copy(
                    g_hbm.at[c, x, pl.ds(y0, YS), pl.ds(0, 128)],
                    buf_a.at[c], sem)
                pltpu.async_copy(
                    g_hbm.at[c, x, pl.ds(y0, YS), pl.ds(128, 32)],
                    buf_b.at[c], sem)
                return 0

            lax.fori_loop(0, C, fire, 0)
            pltpu.make_async_copy(
                g_hbm.at[pl.ds(0, C), 0, pl.ds(0, YS), pl.ds(0, 128)],
                buf_a, sem).wait()
            pltpu.make_async_copy(
                g_hbm.at[pl.ds(0, C), 0, pl.ds(0, YS), pl.ds(128, 32)],
                buf_b, sem).wait()

            def pair_body(jy, _):
                j = jy // YS
                yy = jy % YS
                rbase = yy * (D2 * CPAD // 128)
                for zg in range(10):
                    if zg < 8:
                        v0 = buf_a[2 * j, yy, pl.ds(zg * 16, 16)]
                        v1 = buf_a[2 * j + 1, yy, pl.ds(zg * 16, 16)]
                    else:
                        v0 = buf_b[2 * j, yy, pl.ds((zg - 8) * 16, 16)]
                        v1 = buf_b[2 * j + 1, yy, pl.ds((zg - 8) * 16, 16)]
                    packed = plsc.bitcast(
                        plsc.pack(v0, v1,
                                  format=plsc.PackFormat.INTERLEAVED),
                        jnp.float32)
                    row1 = lane_hi + (rbase + zg * 2)
                    plsc.store_scatter(ob, [row1, lane_lo16 + j], packed)
                    row2 = rp2[zg] + rbase
                    col2 = cp2[zg] + (6 + j)
                    if zg == 0:
                        plsc.store_scatter(ob, [row2, col2], packed, mask=m0)
                    else:
                        plsc.store_scatter(ob, [row2, col2], packed)
                return 0

            lax.fori_loop(0, NPAIR * YS, pair_body, 0)
            row0 = (x * D1 + y0) * (D2 * CPAD // 128)
            pltpu.sync_copy(ob, out_hbm.at[pl.ds(row0, obr), :])
            return 0

        lax.fori_loop(0, xpw * nslab, slab_body, 0)

    return k(grid)


def _build_sc_kernel(n_pts):
    ppw = n_pts // NWORKERS
    nchunks = ppw // CHUNK
    nh = nchunks // 2
    mesh = plsc.VectorSubcoreMesh(core_axis_name="c", subcore_axis_name="s")

    @functools.partial(
        pl.kernel,
        mesh=mesh,
        compiler_params=pltpu.CompilerParams(
            needs_layout_passes=False, use_tc_tiling_on_sc=False),
        out_type=jax.ShapeDtypeStruct((C, n_pts), jnp.float32),
        scratch_types=[
            pltpu.VMEM((8, 16), jnp.float32),
            pltpu.VMEM((3, CHUNK), jnp.float32),
            pltpu.VMEM((2, NIDX), jnp.int32),
            pltpu.VMEM((2, 8 * CHUNK), jnp.float32),
            pltpu.VMEM((2, NIDX, CPAD), jnp.float32),
            pltpu.VMEM((C, CHUNK), jnp.float32),
            pltpu.SemaphoreType.DMA,
            pltpu.SemaphoreType.DMA,
        ],
    )
    def k(q_hbm, table_hbm, consts_hbm, out_hbm,
          consts_v, q_v, idx_v, w_v, rows_v, out_v, sem0, sem1):
        cid = lax.axis_index("c")
        sid = lax.axis_index("s")
        wid = sid * 2 + cid
        base0 = wid * ppw

        pltpu.sync_copy(consts_hbm, consts_v)
        lane = lax.iota(jnp.int32, 16)
        sx = consts_v[0, :]
        sy = consts_v[1, :]
        sz = consts_v[2, :]
        ox = consts_v[3, :]
        oy = consts_v[4, :]
        oz = consts_v[5, :]
        sems = (sem0, sem1)

        def produce(ci, p):
            base = base0 + ci * CHUNK
            pltpu.sync_copy(q_hbm.at[:, pl.ds(base, CHUNK)], q_v)

            def grp_a(g, _):
                p16 = g * 16
                x = q_v[0, pl.ds(p16, 16)]
                y = q_v[1, pl.ds(p16, 16)]
                z = q_v[2, pl.ds(p16, 16)]
                qx = x * sx + ox
                qy = y * sy + oy
                qz = z * sz + oz
                ix = jnp.minimum(jnp.maximum(qx.astype(jnp.int32), 0), D0 - 2)
                iy = jnp.minimum(jnp.maximum(qy.astype(jnp.int32), 0), D1 - 2)
                iz = jnp.minimum(jnp.maximum(qz.astype(jnp.int32), 0), D2 - 2)
                fx = qx - ix.astype(jnp.float32)
                fy = qy - iy.astype(jnp.float32)
                fz = qz - iz.astype(jnp.float32)
                flat = ix * (D1 * D2) + iy * D2 + iz
                gx0 = 1.0 - fx
                gy0 = 1.0 - fy
                gz0 = 1.0 - fz
                wxy = (gx0 * gy0, gx0 * fy, fx * gy0, fx * fy)
                for kk in range(4):
                    dx, dy = kk >> 1, kk & 1
                    corner_off = dx * (D1 * D2) + dy * D2
                    idx_v[p, pl.ds(kk * CHUNK + p16, 16)] = flat + corner_off
                    w_v[p, pl.ds(2 * kk * CHUNK + p16, 16)] = wxy[kk] * gz0
                    w_v[p, pl.ds((2 * kk + 1) * CHUNK + p16, 16)] = \
                        wxy[kk] * fz
                return 0

            lax.fori_loop(0, CHUNK // 16, grp_a, 0)
            for j in range(NIDX // GATHER_SLICE):
                pltpu.async_copy(
                    table_hbm.at[idx_v.at[p, pl.ds(j * GATHER_SLICE,
                                                   GATHER_SLICE)]],
                    rows_v.at[p, pl.ds(j * GATHER_SLICE, GATHER_SLICE), :],
                    sems[p])

        def consume(ci, p):
            base = base0 + ci * CHUNK
            pltpu.make_async_copy(
                table_hbm.at[pl.ds(0, NIDX), :], rows_v.at[p], sems[p]).wait()

            def grp_c(g, _):
                p16 = g * 16
                ws0 = [w_v[p, pl.ds(2 * kk * CHUNK + p16, 16)]
                       for kk in range(4)]
                ws1 = [w_v[p, pl.ds((2 * kk + 1) * CHUNK + p16, 16)]
                       for kk in range(4)]
                rows = [lane + (p16 + kk * CHUNK) for kk in range(4)]
                for j in range(NPAIR):
                    colz0 = jnp.full((16,), j, jnp.int32)
                    colz1 = jnp.full((16,), 6 + j, jnp.int32)
                    acc0 = jnp.zeros((16,), jnp.float32)
                    acc1 = jnp.zeros((16,), jnp.float32)
                    for kk in range(4):
                        g0 = plsc.load_gather(rows_v.at[p], [rows[kk], colz0])
                        g1 = plsc.load_gather(rows_v.at[p], [rows[kk], colz1])
                        a0, b0 = plsc.unpack(
                            plsc.bitcast(g0, jnp.bfloat16),
                            format=plsc.PackFormat.INTERLEAVED)
                        a1, b1 = plsc.unpack(
                            plsc.bitcast(g1, jnp.bfloat16),
                            format=plsc.PackFormat.INTERLEAVED)
                        acc0 = acc0 + ws0[kk] * a0 + ws1[kk] * a1
                        acc1 = acc1 + ws0[kk] * b0 + ws1[kk] * b1
                    out_v[2 * j, pl.ds(p16, 16)] = acc0
                    out_v[2 * j + 1, pl.ds(p16, 16)] = acc1
                return 0

            lax.fori_loop(0, CHUNK // 16, grp_c, 0)
            pltpu.sync_copy(out_v, out_hbm.at[:, pl.ds(base, CHUNK)])

        produce(0, 0)

        def pair_body(i, _):
            ci = 2 * i
            produce(ci + 1, 1)
            consume(ci, 0)

            @pl.when(i < nh - 1)
            def _():
                produce(ci + 2, 0)

            consume(ci + 1, 1)
            return 0

        lax.fori_loop(0, nh, pair_body, 0)

    return k


def kernel(xyz, grid, xyz_min, xyz_max):
    shape = xyz.shape[:-1]
    pts = xyz.reshape(-1, 3)
    n_pts = pts.shape[0]

    table = _repack_table(grid).reshape(NROWS, CPAD)

    sizes = jnp.array([D0 - 1, D1 - 1, D2 - 1], dtype=jnp.float32)
    scale = sizes / (xyz_max - xyz_min)
    off = -xyz_min * scale
    consts = jnp.zeros((8, 16), jnp.float32)
    consts = consts.at[0:3, :].set(jnp.broadcast_to(scale[:, None], (3, 16)))
    consts = consts.at[3:6, :].set(jnp.broadcast_to(off[:, None], (3, 16)))

    q_cm = pts.T
    out_cm = _build_sc_kernel(n_pts)(q_cm, table, consts)
    out = out_cm.T.reshape(*shape, C)
    return out
